# baseline (speedup 1.0000x reference)
import functools

import jax
import jax.numpy as jnp
from jax import lax
from jax.experimental import pallas as pl
from jax.experimental.pallas import tpu as pltpu
from jax.experimental.pallas import tpu_sc as plsc

VOCAB = 100000
EMBED = 1024
WINDOW = 2048
BATCH = 4

NUM_CORES = 2
NUM_SUBCORES = 16
NUM_WORKERS = NUM_CORES * NUM_SUBCORES
POS_PER_WORKER = WINDOW // NUM_WORKERS
CHUNK = 8
NCHUNK = POS_PER_WORKER // CHUNK
LANES = 16
VECS_PER_ROW = EMBED // LANES
NBUF = 3


def _make_kernel():
    mesh = plsc.VectorSubcoreMesh(core_axis_name="c", subcore_axis_name="s")

    @functools.partial(
        pl.kernel,
        mesh=mesh,
        out_type=jax.ShapeDtypeStruct((BATCH, WINDOW, EMBED), jnp.float32),
        scratch_types=[
            pltpu.VMEM((BATCH * POS_PER_WORKER,), jnp.int32),
            pltpu.VMEM((NBUF, CHUNK, EMBED), jnp.float32),
            pltpu.VMEM((NBUF, BATCH, CHUNK, EMBED), jnp.float32),
            pltpu.SemaphoreType.DMA,
            pltpu.SemaphoreType.DMA,
            pltpu.SemaphoreType.DMA,
            pltpu.SemaphoreType.DMA,
            pltpu.SemaphoreType.DMA,
            pltpu.SemaphoreType.DMA,
            pltpu.SemaphoreType.DMA,
            pltpu.SemaphoreType.DMA,
            pltpu.SemaphoreType.DMA,
            pltpu.SemaphoreType.DMA,
        ],
    )
    def emb_kernel(tokens_hbm, ttab_hbm, ptab_hbm, out_hbm,
                   idx_v, pos_v, tok_v,
                   gsem0, gsem1, gsem2, ssem0, ssem1, ssem2,
                   psem0, psem1, psem2, isem):
        wid = lax.axis_index("s") * NUM_CORES + lax.axis_index("c")
        pstart = wid * POS_PER_WORKER
        gsems = (gsem0, gsem1, gsem2)
        ssems = (ssem0, ssem1, ssem2)
        psems = (psem0, psem1, psem2)

        def issue_pos(c, s):
            return pltpu.async_copy(
                ptab_hbm.at[pl.ds(pstart + c * CHUNK, CHUNK)],
                pos_v.at[s], psems[s])

        issue_pos(0, 0)
        issue_pos(1, 1)

        idx_cps = []
        for b in range(BATCH):
            idx_cps.append(pltpu.async_copy(
                tokens_hbm.at[pl.ds(b * WINDOW + pstart, POS_PER_WORKER)],
                idx_v.at[pl.ds(b * POS_PER_WORKER, POS_PER_WORKER)], isem))
        del idx_cps
        pltpu.make_async_copy(
            tokens_hbm.at[pl.ds(0, BATCH * POS_PER_WORKER)], idx_v,
            isem).wait()

        def issue_gathers(c, s):
            cps = []
            for b in range(BATCH):
                idx_sl = idx_v.at[pl.ds(b * POS_PER_WORKER + c * CHUNK, CHUNK)]
                cps.append(pltpu.async_copy(
                    ttab_hbm.at[idx_sl], tok_v.at[s, b], gsems[s]))
            return cps

        def wait_gathers(s):
            pltpu.make_async_copy(
                out_hbm.at[:, pl.ds(0, CHUNK), :], tok_v.at[s],
                gsems[s]).wait()

        def wait_pos(s):
            pltpu.make_async_copy(
                ptab_hbm.at[pl.ds(0, CHUNK)], pos_v.at[s], psems[s]).wait()

        def do_add(s):
            def body(r, carry):
                for j in range(VECS_PER_ROW):
                    sl = pl.ds(j * LANES, LANES)
                    p = pos_v[s, r, sl]
                    for b in range(BATCH):
                        tok_v[s, b, r, sl] = tok_v[s, b, r, sl] + p
                return carry

            lax.fori_loop(0, CHUNK, body, 0)

        def issue_store(c, s):
            return pltpu.async_copy(
                tok_v.at[s],
                out_hbm.at[:, pl.ds(pstart + c * CHUNK, CHUNK), :],
                ssems[s])

        def wait_store(s):
            pltpu.make_async_copy(
                tok_v.at[s], out_hbm.at[:, pl.ds(0, CHUNK), :],
                ssems[s]).wait()

        def process(c, s):
            s2 = (s + 2) % NBUF

            @pl.when(c + 2 < NCHUNK)
            def _():
                issue_pos(c + 2, s2)

                @pl.when(c >= 1)
                def _():
                    wait_store(s2)

                issue_gathers(c + 2, s2)

            wait_pos(s)
            wait_gathers(s)
            do_add(s)
            issue_store(c, s)

        issue_gathers(0, 0)
        issue_gathers(1, 1)

        def triple(t, carry):
            c = t * NBUF
            process(c, 0)
            process(c + 1, 1)

            @pl.when(c + 2 < NCHUNK)
            def _():
                process(c + 2, 2)

            return carry

        lax.fori_loop(0, (NCHUNK + NBUF - 1) // NBUF, triple, 0)

        for s in (2, 0, 1):
            wait_store(s)

    return emb_kernel


_EMB_KERNEL = _make_kernel()


def kernel(tokens, token_table, position_table):
    flat_tokens = tokens.reshape(BATCH * WINDOW).astype(jnp.int32)
    return _EMB_KERNEL(flat_tokens, token_table, position_table)
